# Initial kernel scaffold; baseline (speedup 1.0000x reference)
import jax
import jax.numpy as jnp
from jax.experimental import pallas as pl


def kernel(x, W, b):
    raise NotImplementedError("write your pallas kernel here")


# speedup vs baseline: 1.8085x; 1.8085x over previous
import functools

import jax
import jax.numpy as jnp
from jax.experimental import pallas as pl
from jax.experimental.pallas import tpu as pltpu

NUM_EXPERTS = 64
TOP_K = 2
ALPHA = 10.0


def _gating_kernel(x_ref, wt_ref, b_ref, out_ref):
    logits = jax.lax.dot_general(
        x_ref[...], wt_ref[...],
        dimension_numbers=(((1,), (0,)), ((), ())),
        preferred_element_type=jnp.float32,
        precision=jax.lax.Precision.HIGHEST,
    ) + b_ref[...]

    neg_inf = jnp.float32(-jnp.inf)
    m1 = jnp.max(logits, axis=1, keepdims=True)
    idx = jnp.argmax(logits, axis=1)[:, None]
    lanes = jax.lax.broadcasted_iota(jnp.int32, logits.shape, 1)
    m2 = jnp.max(jnp.where(lanes == idx, neg_inf, logits), axis=1,
                 keepdims=True)

    mask = logits < m2

    e = jnp.exp(logits - m1)
    sx = e / jnp.sum(e, axis=1, keepdims=True)

    out = jnp.where(mask,
                    ALPHA * jnp.log(sx + 1.0),
                    ALPHA * (jnp.exp(sx) - 1.0))

    om = jnp.max(out, axis=1, keepdims=True)
    g = jnp.exp(out - om)
    out_ref[...] = g / jnp.sum(g, axis=1, keepdims=True)


@functools.partial(jax.jit, static_argnames=("tile_n",))
def _run(x, wt, b2d, tile_n):
    n = x.shape[0]
    grid = (n // tile_n,)
    return pl.pallas_call(
        _gating_kernel,
        grid=grid,
        in_specs=[
            pl.BlockSpec((tile_n, x.shape[1]), lambda i: (i, 0)),
            pl.BlockSpec(wt.shape, lambda i: (0, 0)),
            pl.BlockSpec(b2d.shape, lambda i: (0, 0)),
        ],
        out_specs=pl.BlockSpec((tile_n, NUM_EXPERTS), lambda i: (i, 0)),
        out_shape=jax.ShapeDtypeStruct((n, NUM_EXPERTS), jnp.float32),
        compiler_params=pltpu.CompilerParams(
            dimension_semantics=("arbitrary",),
        ),
    )(x, wt, b2d)


def kernel(x, W, b):
    wt = W.T
    b2d = b.reshape(1, NUM_EXPERTS)
    return _run(x, wt, b2d, tile_n=2048)


# speedup vs baseline: 3.5630x; 1.9702x over previous
import functools

import jax
import jax.numpy as jnp
from jax.experimental import pallas as pl
from jax.experimental.pallas import tpu as pltpu

NUM_EXPERTS = 64
TOP_K = 2
ALPHA = 10.0


def _gating_kernel(x_ref, wt_ref, b_ref, out_ref):
    logits = jax.lax.dot_general(
        x_ref[...], wt_ref[...],
        dimension_numbers=(((1,), (0,)), ((), ())),
        preferred_element_type=jnp.float32,
        precision=jax.lax.Precision.DEFAULT,
    ) + b_ref[...]

    neg_inf = jnp.float32(-jnp.inf)
    m1 = jnp.max(logits, axis=1, keepdims=True)
    idx = jnp.argmax(logits, axis=1)[:, None]
    lanes = jax.lax.broadcasted_iota(jnp.int32, logits.shape, 1)
    m2 = jnp.max(jnp.where(lanes == idx, neg_inf, logits), axis=1,
                 keepdims=True)

    mask = logits < m2

    e = jnp.exp(logits - m1)
    sx = e / jnp.sum(e, axis=1, keepdims=True)

    out = jnp.where(mask,
                    ALPHA * jnp.log(sx + 1.0),
                    ALPHA * (jnp.exp(sx) - 1.0))

    om = jnp.max(out, axis=1, keepdims=True)
    g = jnp.exp(out - om)
    out_ref[...] = g / jnp.sum(g, axis=1, keepdims=True)


@functools.partial(jax.jit, static_argnames=("tile_n",))
def _run(x, wt, b2d, tile_n):
    n = x.shape[0]
    grid = (n // tile_n,)
    return pl.pallas_call(
        _gating_kernel,
        grid=grid,
        in_specs=[
            pl.BlockSpec((tile_n, x.shape[1]), lambda i: (i, 0)),
            pl.BlockSpec(wt.shape, lambda i: (0, 0)),
            pl.BlockSpec(b2d.shape, lambda i: (0, 0)),
        ],
        out_specs=pl.BlockSpec((tile_n, NUM_EXPERTS), lambda i: (i, 0)),
        out_shape=jax.ShapeDtypeStruct((n, NUM_EXPERTS), jnp.float32),
        compiler_params=pltpu.CompilerParams(
            dimension_semantics=("arbitrary",),
        ),
    )(x, wt, b2d)


def kernel(x, W, b):
    wt = W.T
    b2d = b.reshape(1, NUM_EXPERTS)
    return _run(x, wt, b2d, tile_n=2048)


# speedup vs baseline: 3.8103x; 1.0694x over previous
import functools

import jax
import jax.numpy as jnp
from jax.experimental import pallas as pl
from jax.experimental.pallas import tpu as pltpu

NUM_EXPERTS = 64
TOP_K = 2
ALPHA = 10.0


def _gating_kernel(x_ref, wt_ref, b_ref, out_ref):
    logits = jax.lax.dot_general(
        x_ref[...], wt_ref[...],
        dimension_numbers=(((1,), (0,)), ((), ())),
        preferred_element_type=jnp.float32,
        precision=jax.lax.Precision.DEFAULT,
    ) + b_ref[...]

    neg_inf = jnp.float32(-jnp.inf)
    m1 = jnp.max(logits, axis=1, keepdims=True)
    idx = jnp.argmax(logits, axis=1)[:, None]
    lanes = jax.lax.broadcasted_iota(jnp.int32, logits.shape, 1)
    m2 = jnp.max(jnp.where(lanes == idx, neg_inf, logits), axis=1,
                 keepdims=True)

    mask = logits < m2

    e = jnp.exp(logits - m1)
    sx = e / jnp.sum(e, axis=1, keepdims=True)

    out = jnp.where(mask,
                    ALPHA * jnp.log(sx + 1.0),
                    ALPHA * (jnp.exp(sx) - 1.0))

    om = jnp.max(out, axis=1, keepdims=True)
    g = jnp.exp(out - om)
    out_ref[...] = g / jnp.sum(g, axis=1, keepdims=True)


@functools.partial(jax.jit, static_argnames=("tile_n",))
def _run(x, wt, b2d, tile_n):
    n = x.shape[0]
    grid = (n // tile_n,)
    return pl.pallas_call(
        _gating_kernel,
        grid=grid,
        in_specs=[
            pl.BlockSpec((tile_n, x.shape[1]), lambda i: (i, 0)),
            pl.BlockSpec(wt.shape, lambda i: (0, 0)),
            pl.BlockSpec(b2d.shape, lambda i: (0, 0)),
        ],
        out_specs=pl.BlockSpec((tile_n, NUM_EXPERTS), lambda i: (i, 0)),
        out_shape=jax.ShapeDtypeStruct((n, NUM_EXPERTS), jnp.float32),
        compiler_params=pltpu.CompilerParams(
            dimension_semantics=("arbitrary",),
        ),
    )(x, wt, b2d)


def kernel(x, W, b):
    wt = W.T
    b2d = b.reshape(1, NUM_EXPERTS)
    return _run(x, wt, b2d, tile_n=4096)
